# speedup vs baseline: 8.1149x; 8.1149x over previous
import functools

import jax
import jax.numpy as jnp
from jax import lax
from jax.experimental import pallas as pl
from jax.experimental.pallas import tpu as pltpu
from jax.experimental.pallas import tpu_sc as plsc

D = 128
N_WORKERS = 32
CHUNK = 512


def _emb_kernel(n_total):
    per_w = n_total // N_WORKERS
    n_chunks = per_w // CHUNK
    mesh = plsc.VectorSubcoreMesh(core_axis_name="c", subcore_axis_name="s")

    @functools.partial(
        pl.kernel,
        mesh=mesh,
        out_type=jax.ShapeDtypeStruct((n_total, D), jnp.float32),
        scratch_types=[
            pltpu.VMEM((CHUNK,), jnp.int32),
            pltpu.VMEM((CHUNK, D), jnp.float32),
            pltpu.SemaphoreType.DMA,
        ],
    )
    def k(idx_hbm, tbl_hbm, out_hbm, idx_v, rows_v, sem):
        wid = lax.axis_index("s") * 2 + lax.axis_index("c")
        base = wid * per_w

        def body(i, carry):
            off = base + i * CHUNK
            pltpu.sync_copy(idx_hbm.at[pl.ds(off, CHUNK)], idx_v)
            pltpu.async_copy(tbl_hbm.at[idx_v], rows_v, sem).wait()
            pltpu.sync_copy(rows_v, out_hbm.at[pl.ds(off, CHUNK)])
            return carry

        lax.fori_loop(0, n_chunks, body, 0)

    return k


def kernel(x, weight):
    b, s = x.shape
    n_total = b * s
    idx = x.reshape(n_total).astype(jnp.int32)
    out = _emb_kernel(n_total)(idx, weight)
    return out.reshape(b, s, weight.shape[1])


# speedup vs baseline: 9.1967x; 1.1333x over previous
import functools

import jax
import jax.numpy as jnp
from jax import lax
from jax.experimental import pallas as pl
from jax.experimental.pallas import tpu as pltpu
from jax.experimental.pallas import tpu_sc as plsc

D = 128
N_WORKERS = 32
CHUNK = 400
NBUF = 2


def _emb_kernel(n_total):
    per_w = n_total // N_WORKERS
    n_chunks = per_w // CHUNK
    n_pairs = n_chunks // NBUF
    mesh = plsc.VectorSubcoreMesh(core_axis_name="c", subcore_axis_name="s")

    @functools.partial(
        pl.kernel,
        mesh=mesh,
        out_type=jax.ShapeDtypeStruct((n_total, D), jnp.float32),
        scratch_types=[
            pltpu.VMEM((CHUNK,), jnp.int32),
            pltpu.VMEM((CHUNK,), jnp.int32),
            pltpu.VMEM((NBUF, CHUNK, D), jnp.float32),
            pltpu.SemaphoreType.DMA,
            pltpu.SemaphoreType.DMA,
            pltpu.SemaphoreType.DMA,
            pltpu.SemaphoreType.DMA,
        ],
    )
    def k(idx_hbm, tbl_hbm, out_hbm, idx0, idx1, rows_v, g0, g1, s0, s1):
        idxb = (idx0, idx1)
        gsem = (g0, g1)
        ssem = (s0, s1)
        wid = lax.axis_index("s") * 2 + lax.axis_index("c")
        base = wid * per_w

        for b in range(NBUF):
            pltpu.sync_copy(idx_hbm.at[pl.ds(base + b * CHUNK, CHUNK)], idxb[b])
            pltpu.async_copy(tbl_hbm.at[idxb[b]], rows_v.at[b], gsem[b])

        def body(g, carry):
            for b in range(NBUF):
                c = g * NBUF + b
                pltpu.make_async_copy(
                    tbl_hbm.at[idxb[b]], rows_v.at[b], gsem[b]
                ).wait()
                out_slc = out_hbm.at[pl.ds(base + c * CHUNK, CHUNK)]
                pltpu.async_copy(rows_v.at[b], out_slc, ssem[b])

                @pl.when(c + NBUF < n_chunks)
                def _():
                    pltpu.make_async_copy(rows_v.at[b], out_slc, ssem[b]).wait()
                    nxt = base + (c + NBUF) * CHUNK
                    pltpu.sync_copy(idx_hbm.at[pl.ds(nxt, CHUNK)], idxb[b])
                    pltpu.async_copy(tbl_hbm.at[idxb[b]], rows_v.at[b], gsem[b])

            return carry

        lax.fori_loop(0, n_pairs, body, 0)

        for b in range(NBUF):
            c = (n_pairs - 1) * NBUF + b
            out_slc = out_hbm.at[pl.ds(base + c * CHUNK, CHUNK)]
            pltpu.make_async_copy(rows_v.at[b], out_slc, ssem[b]).wait()

    return k


def kernel(x, weight):
    b, s = x.shape
    n_total = b * s
    idx = x.reshape(n_total).astype(jnp.int32)
    out = _emb_kernel(n_total)(idx, weight)
    return out.reshape(b, s, weight.shape[1])
